# baseline (speedup 1.0000x reference)
import functools

import jax
import jax.numpy as jnp
from jax import lax
from jax.experimental import pallas as pl
from jax.experimental.pallas import tpu as pltpu
from jax.experimental.pallas import tpu_sc as plsc

E = 64
T = 128
D = 1024
F = 1024
G = 2
NG = E // G
NBUF = 2
PF = 1

SC_NC = 2
SC_NS = 16
SC_L = 16
SC_WORKERS = 8
TPW = T // SC_WORKERS
NV = E // SC_L

_NEG = -3.4028235e38


def _routing_sc_kernel(logits_hbm, i1_hbm, i2_hbm, s1_hbm, s2_hbm,
                       logits_v, i1_v, i2_v, s1_v, s2_v):
    wid = lax.axis_index("s") * SC_NC + lax.axis_index("c")

    @pl.when(wid < SC_WORKERS)
    def _work():
        base = wid * TPW
        pltpu.sync_copy(logits_hbm.at[pl.ds(base, TPW)], logits_v)
        lane = jnp.arange(SC_L, dtype=jnp.int32)
        acc_i1 = jnp.zeros((SC_L,), jnp.int32)
        acc_i2 = jnp.zeros((SC_L,), jnp.int32)
        acc_s1 = jnp.zeros((SC_L,), jnp.float32)
        acc_s2 = jnp.zeros((SC_L,), jnp.float32)
        for t in range(TPW):
            vs = [logits_v[t, pl.ds(j * SC_L, SC_L)] for j in range(NV)]
            iotas = [lane + SC_L * j for j in range(NV)]
            m = vs[0]
            for v in vs[1:]:
                m = jnp.maximum(m, v)
            l1 = jnp.max(m)
            i1 = jnp.int32(E)
            for v, io in zip(vs, iotas):
                i1 = jnp.minimum(i1, jnp.min(jnp.where(v == l1, io, E)))
            ws = [jnp.where(io == i1, _NEG, v) for v, io in zip(vs, iotas)]
            m2 = ws[0]
            for v in ws[1:]:
                m2 = jnp.maximum(m2, v)
            l2 = jnp.max(m2)
            i2 = jnp.int32(E)
            for v, io in zip(ws, iotas):
                i2 = jnp.minimum(i2, jnp.min(jnp.where(v == l2, io, E)))
            dv = jnp.full((SC_L,), l2 - l1, dtype=jnp.float32)
            s1v = 1.0 / (1.0 + jnp.exp(dv))
            tmask = lane == t
            acc_i1 = jnp.where(tmask, jnp.full((SC_L,), i1), acc_i1)
            acc_i2 = jnp.where(tmask, jnp.full((SC_L,), i2), acc_i2)
            acc_s1 = jnp.where(tmask, s1v, acc_s1)
            acc_s2 = jnp.where(tmask, 1.0 - s1v, acc_s2)
        i1_v[...] = acc_i1
        i2_v[...] = acc_i2
        s1_v[...] = acc_s1
        s2_v[...] = acc_s2
        pltpu.sync_copy(i1_v, i1_hbm.at[pl.ds(base, TPW)])
        pltpu.sync_copy(i2_v, i2_hbm.at[pl.ds(base, TPW)])
        pltpu.sync_copy(s1_v, s1_hbm.at[pl.ds(base, TPW)])
        pltpu.sync_copy(s2_v, s2_hbm.at[pl.ds(base, TPW)])


_routing_sc = pl.kernel(
    _routing_sc_kernel,
    out_type=(
        jax.ShapeDtypeStruct((T,), jnp.int32),
        jax.ShapeDtypeStruct((T,), jnp.int32),
        jax.ShapeDtypeStruct((T,), jnp.float32),
        jax.ShapeDtypeStruct((T,), jnp.float32),
    ),
    mesh=plsc.VectorSubcoreMesh(core_axis_name="c", subcore_axis_name="s"),
    compiler_params=pltpu.CompilerParams(needs_layout_passes=False),
    scratch_types=[
        pltpu.VMEM((TPW, E), jnp.float32),
        pltpu.VMEM((TPW,), jnp.int32),
        pltpu.VMEM((TPW,), jnp.int32),
        pltpu.VMEM((TPW,), jnp.float32),
        pltpu.VMEM((TPW,), jnp.float32),
    ],
)


def _moe_kernel(x_ref, i1_ref, i2_ref, s1_ref, s2_ref,
                w1_hbm, w3_hbm, w2_hbm, out_ref,
                w1_buf, w3_buf, w2_buf, w1_sem, w3_sem, w2_sem):
    def _copies(g):
        slot = jax.lax.rem(g, NBUF)
        src = pl.ds(g * G, G)
        return (
            pltpu.make_async_copy(w1_hbm.at[src], w1_buf.at[slot], w1_sem.at[slot]),
            pltpu.make_async_copy(w3_hbm.at[src], w3_buf.at[slot], w3_sem.at[slot]),
            pltpu.make_async_copy(w2_hbm.at[src], w2_buf.at[slot], w2_sem.at[slot]),
        )

    def _issue(g):
        for c in _copies(g):
            c.start()

    for g in range(PF):
        _issue(g)

    out_ref[...] = jnp.zeros((T, D), jnp.float32)
    xb = x_ref[...]

    def _body(g, carry):
        @pl.when(g + PF < NG)
        def _prefetch():
            _issue(g + PF)

        c1, c3, c2 = _copies(g)
        slot = jax.lax.rem(g, NBUF)
        c1.wait()
        gs = [jax.lax.dot_general(
                  xb, w1_buf[slot, sub], (((1,), (1,)), ((), ())),
                  preferred_element_type=jnp.float32) for sub in range(G)]
        c3.wait()
        hs = []
        for sub in range(G):
            u = jax.lax.dot_general(
                xb, w3_buf[slot, sub], (((1,), (1,)), ((), ())),
                preferred_element_type=jnp.float32)
            gg = gs[sub]
            h = (gg * jax.nn.sigmoid(gg)) * u
            e = g * G + sub
            scale = (jnp.where(i1_ref[...] == e, s1_ref[...], 0.0)
                     + jnp.where(i2_ref[...] == e, s2_ref[...], 0.0))
            hs.append(h * scale)
        c2.wait()
        acc = None
        for sub in range(G):
            y = jax.lax.dot_general(
                hs[sub], w2_buf[slot, sub], (((1,), (1,)), ((), ())),
                preferred_element_type=jnp.float32)
            acc = y if acc is None else acc + y
        out_ref[...] += acc
        return carry

    jax.lax.fori_loop(0, NG, _body, 0, unroll=False)


@jax.jit
def kernel(x, router_logits, w1, w3, w2):
    i1, i2, s1, s2 = _routing_sc(router_logits)
    return pl.pallas_call(
        _moe_kernel,
        in_specs=[
            pl.BlockSpec((T, D), lambda: (0, 0)),
            pl.BlockSpec((T, 1), lambda: (0, 0)),
            pl.BlockSpec((T, 1), lambda: (0, 0)),
            pl.BlockSpec((T, 1), lambda: (0, 0)),
            pl.BlockSpec((T, 1), lambda: (0, 0)),
            pl.BlockSpec(memory_space=pltpu.MemorySpace.HBM),
            pl.BlockSpec(memory_space=pltpu.MemorySpace.HBM),
            pl.BlockSpec(memory_space=pltpu.MemorySpace.HBM),
        ],
        out_specs=pl.BlockSpec((T, D), lambda: (0, 0)),
        out_shape=jax.ShapeDtypeStruct((T, D), jnp.float32),
        scratch_shapes=[
            pltpu.VMEM((NBUF, G, F, D), jnp.float32),
            pltpu.VMEM((NBUF, G, F, D), jnp.float32),
            pltpu.VMEM((NBUF, G, D, F), jnp.float32),
            pltpu.SemaphoreType.DMA((NBUF,)),
            pltpu.SemaphoreType.DMA((NBUF,)),
            pltpu.SemaphoreType.DMA((NBUF,)),
        ],
        compiler_params=pltpu.CompilerParams(
            vmem_limit_bytes=60000 * 1024,
        ),
    )(x, i1.reshape(T, 1), i2.reshape(T, 1),
      s1.reshape(T, 1), s2.reshape(T, 1), w1, w3, w2)


# speedup vs baseline: 1.0233x; 1.0233x over previous
import functools

import jax
import jax.numpy as jnp
from jax import lax
from jax.experimental import pallas as pl
from jax.experimental.pallas import tpu as pltpu
from jax.experimental.pallas import tpu_sc as plsc

E = 64
T = 128
D = 1024
F = 1024
G = 2
NG = E // G
NBUF = 2
PF = 1

SC_NC = 2
SC_NS = 16
SC_L = 16
SC_WORKERS = SC_NC * SC_NS
TPW = T // SC_WORKERS
NV = E // SC_L

_NEG = -3.4028235e38


def _routing_sc_kernel(logits_hbm, route_hbm, logits_v, route_v):
    wid = lax.axis_index("s") * SC_NC + lax.axis_index("c")
    base = wid * TPW
    pltpu.sync_copy(logits_hbm.at[pl.ds(base, TPW)], logits_v)
    lane = jnp.arange(SC_L, dtype=jnp.int32)
    acc = jnp.zeros((SC_L,), jnp.float32)
    for t in range(TPW):
        vs = [logits_v[t, pl.ds(j * SC_L, SC_L)] for j in range(NV)]
        iotas = [lane + SC_L * j for j in range(NV)]
        m = vs[0]
        for v in vs[1:]:
            m = jnp.maximum(m, v)
        l1 = jnp.max(m)
        i1 = jnp.int32(E)
        for v, io in zip(vs, iotas):
            i1 = jnp.minimum(i1, jnp.min(jnp.where(v == l1, io, E)))
        ws = [jnp.where(io == i1, _NEG, v) for v, io in zip(vs, iotas)]
        m2 = ws[0]
        for v in ws[1:]:
            m2 = jnp.maximum(m2, v)
        l2 = jnp.max(m2)
        i2 = jnp.int32(E)
        for v, io in zip(ws, iotas):
            i2 = jnp.minimum(i2, jnp.min(jnp.where(v == l2, io, E)))
        dv = jnp.full((SC_L,), l2 - l1, dtype=jnp.float32)
        s1v = 1.0 / (1.0 + jnp.exp(dv))
        acc = jnp.where(lane == 4 * t, jnp.full((SC_L,), i1.astype(jnp.float32)), acc)
        acc = jnp.where(lane == 4 * t + 1, jnp.full((SC_L,), i2.astype(jnp.float32)), acc)
        acc = jnp.where(lane == 4 * t + 2, s1v, acc)
        acc = jnp.where(lane == 4 * t + 3, 1.0 - s1v, acc)
    route_v[...] = acc
    pltpu.sync_copy(route_v, route_hbm.at[pl.ds(base * 4, SC_L)])


_routing_sc = pl.kernel(
    _routing_sc_kernel,
    out_type=jax.ShapeDtypeStruct((T * 4,), jnp.float32),
    mesh=plsc.VectorSubcoreMesh(core_axis_name="c", subcore_axis_name="s"),
    compiler_params=pltpu.CompilerParams(needs_layout_passes=False),
    scratch_types=[
        pltpu.VMEM((TPW, E), jnp.float32),
        pltpu.VMEM((SC_L,), jnp.float32),
    ],
)


def _moe_kernel(x_ref, route_ref,
                w1_hbm, w3_hbm, w2_hbm, out_ref,
                w1_buf, w3_buf, w2_buf, w1_sem, w3_sem, w2_sem):
    def _copies(g):
        slot = jax.lax.rem(g, NBUF)
        src = pl.ds(g * G, G)
        return (
            pltpu.make_async_copy(w1_hbm.at[src], w1_buf.at[slot], w1_sem.at[slot]),
            pltpu.make_async_copy(w3_hbm.at[src], w3_buf.at[slot], w3_sem.at[slot]),
            pltpu.make_async_copy(w2_hbm.at[src], w2_buf.at[slot], w2_sem.at[slot]),
        )

    def _issue(g):
        for c in _copies(g):
            c.start()

    for g in range(PF):
        _issue(g)

    out_ref[...] = jnp.zeros((T, D), jnp.float32)
    xb = x_ref[...]

    def _body(g, carry):
        @pl.when(g + PF < NG)
        def _prefetch():
            _issue(g + PF)

        c1, c3, c2 = _copies(g)
        slot = jax.lax.rem(g, NBUF)
        c1.wait()
        gs = [jax.lax.dot_general(
                  xb, w1_buf[slot, sub], (((1,), (1,)), ((), ())),
                  preferred_element_type=jnp.float32) for sub in range(G)]
        c3.wait()
        hs = []
        for sub in range(G):
            u = jax.lax.dot_general(
                xb, w3_buf[slot, sub], (((1,), (1,)), ((), ())),
                preferred_element_type=jnp.float32)
            gg = gs[sub]
            h = (gg * jax.nn.sigmoid(gg)) * u
            e = g * G + sub
            ef = e.astype(jnp.float32)
            scale = (jnp.where(route_ref[:, 0:1] == ef, route_ref[:, 2:3], 0.0)
                     + jnp.where(route_ref[:, 1:2] == ef, route_ref[:, 3:4], 0.0))
            hs.append(h * scale)
        c2.wait()
        acc = None
        for sub in range(G):
            y = jax.lax.dot_general(
                hs[sub], w2_buf[slot, sub], (((1,), (1,)), ((), ())),
                preferred_element_type=jnp.float32)
            acc = y if acc is None else acc + y
        out_ref[...] += acc
        return carry

    jax.lax.fori_loop(0, NG, _body, 0, unroll=False)


@jax.jit
def kernel(x, router_logits, w1, w3, w2):
    route = _routing_sc(router_logits).reshape(T, 4)
    return pl.pallas_call(
        _moe_kernel,
        in_specs=[
            pl.BlockSpec((T, D), lambda: (0, 0)),
            pl.BlockSpec((T, 4), lambda: (0, 0)),
            pl.BlockSpec(memory_space=pltpu.MemorySpace.HBM),
            pl.BlockSpec(memory_space=pltpu.MemorySpace.HBM),
            pl.BlockSpec(memory_space=pltpu.MemorySpace.HBM),
        ],
        out_specs=pl.BlockSpec((T, D), lambda: (0, 0)),
        out_shape=jax.ShapeDtypeStruct((T, D), jnp.float32),
        scratch_shapes=[
            pltpu.VMEM((NBUF, G, F, D), jnp.float32),
            pltpu.VMEM((NBUF, G, F, D), jnp.float32),
            pltpu.VMEM((NBUF, G, D, F), jnp.float32),
            pltpu.SemaphoreType.DMA((NBUF,)),
            pltpu.SemaphoreType.DMA((NBUF,)),
            pltpu.SemaphoreType.DMA((NBUF,)),
        ],
        compiler_params=pltpu.CompilerParams(
            vmem_limit_bytes=60000 * 1024,
        ),
    )(x, route, w1, w3, w2)


# speedup vs baseline: 1.0294x; 1.0060x over previous
import functools

import jax
import jax.numpy as jnp
from jax import lax
from jax.experimental import pallas as pl
from jax.experimental.pallas import tpu as pltpu
from jax.experimental.pallas import tpu_sc as plsc

E = 64
T = 128
D = 1024
F = 1024
G = 2
NG = E // G
NBUF = 2
PF = 1

SC_NC = 2
SC_NS = 16
SC_L = 16
SC_WORKERS = SC_NC * SC_NS
TPW = T // SC_WORKERS
NV = E // SC_L

_NEG = -3.4028235e38


def _routing_sc_kernel(logits_hbm, route_hbm, logits_v, route_v):
    wid = lax.axis_index("s") * SC_NC + lax.axis_index("c")
    base = wid * TPW
    pltpu.sync_copy(logits_hbm.at[pl.ds(base, TPW)], logits_v)
    lane = jnp.arange(SC_L, dtype=jnp.int32)
    acc = jnp.zeros((SC_L,), jnp.float32)
    for t in range(TPW):
        vs = [logits_v[t, pl.ds(j * SC_L, SC_L)] for j in range(NV)]
        iotas = [lane + SC_L * j for j in range(NV)]
        m = vs[0]
        for v in vs[1:]:
            m = jnp.maximum(m, v)
        l1 = jnp.max(m)
        i1 = jnp.int32(E)
        for v, io in zip(vs, iotas):
            i1 = jnp.minimum(i1, jnp.min(jnp.where(v == l1, io, E)))
        ws = [jnp.where(io == i1, _NEG, v) for v, io in zip(vs, iotas)]
        m2 = ws[0]
        for v in ws[1:]:
            m2 = jnp.maximum(m2, v)
        l2 = jnp.max(m2)
        i2 = jnp.int32(E)
        for v, io in zip(ws, iotas):
            i2 = jnp.minimum(i2, jnp.min(jnp.where(v == l2, io, E)))
        dv = jnp.full((SC_L,), l2 - l1, dtype=jnp.float32)
        s1v = 1.0 / (1.0 + jnp.exp(dv))
        acc = jnp.where(lane == 4 * t, jnp.full((SC_L,), i1.astype(jnp.float32)), acc)
        acc = jnp.where(lane == 4 * t + 1, jnp.full((SC_L,), i2.astype(jnp.float32)), acc)
        acc = jnp.where(lane == 4 * t + 2, s1v, acc)
        acc = jnp.where(lane == 4 * t + 3, 1.0 - s1v, acc)
    route_v[...] = acc
    pltpu.sync_copy(route_v, route_hbm.at[pl.ds(base * 4, SC_L)])


_routing_sc = pl.kernel(
    _routing_sc_kernel,
    out_type=jax.ShapeDtypeStruct((T * 4,), jnp.float32),
    mesh=plsc.VectorSubcoreMesh(core_axis_name="c", subcore_axis_name="s"),
    compiler_params=pltpu.CompilerParams(needs_layout_passes=False),
    scratch_types=[
        pltpu.VMEM((TPW, E), jnp.float32),
        pltpu.VMEM((SC_L,), jnp.float32),
    ],
)


def _moe_kernel(x_ref, route_ref,
                w1_hbm, w3_hbm, w2_hbm, out_ref,
                w1_buf, w3_buf, w2_buf, w1_sem, w3_sem, w2_sem):
    def _copies(g):
        slot = jax.lax.rem(g, NBUF)
        src = pl.ds(g * G, G)
        return (
            pltpu.make_async_copy(w1_hbm.at[src], w1_buf.at[slot], w1_sem.at[slot]),
            pltpu.make_async_copy(w3_hbm.at[src], w3_buf.at[slot], w3_sem.at[slot]),
            pltpu.make_async_copy(w2_hbm.at[src], w2_buf.at[slot], w2_sem.at[slot]),
        )

    def _issue(g):
        for c in _copies(g):
            c.start()

    for g in range(PF):
        _issue(g)

    out_ref[...] = jnp.zeros((T, D), jnp.float32)
    xb = x_ref[...]
    route = route_ref[...]
    i1c, i2c, s1c, s2c = (route[:, 0:1], route[:, 1:2],
                          route[:, 2:3], route[:, 3:4])

    def _body(g, carry):
        @pl.when(g + PF < NG)
        def _prefetch():
            _issue(g + PF)

        c1, c3, c2 = _copies(g)
        slot = jax.lax.rem(g, NBUF)
        c1.wait()
        gs = [jax.lax.dot_general(
                  xb, w1_buf[slot, sub], (((1,), (1,)), ((), ())),
                  preferred_element_type=jnp.float32) for sub in range(G)]
        c3.wait()
        hs = []
        for sub in range(G):
            u = jax.lax.dot_general(
                xb, w3_buf[slot, sub], (((1,), (1,)), ((), ())),
                preferred_element_type=jnp.float32)
            gg = gs[sub]
            h = (gg * jax.nn.sigmoid(gg)) * u
            e = g * G + sub
            ef = e.astype(jnp.float32)
            scale = (jnp.where(i1c == ef, s1c, 0.0)
                     + jnp.where(i2c == ef, s2c, 0.0))
            hs.append(h * scale)
        c2.wait()
        acc = None
        for sub in range(G):
            y = jax.lax.dot_general(
                hs[sub], w2_buf[slot, sub], (((1,), (1,)), ((), ())),
                preferred_element_type=jnp.float32)
            acc = y if acc is None else acc + y
        out_ref[...] += acc
        return carry

    jax.lax.fori_loop(0, NG, _body, 0, unroll=False)


@jax.jit
def kernel(x, router_logits, w1, w3, w2):
    route = _routing_sc(router_logits).reshape(T, 4)
    return pl.pallas_call(
        _moe_kernel,
        in_specs=[
            pl.BlockSpec((T, D), lambda: (0, 0)),
            pl.BlockSpec((T, 4), lambda: (0, 0)),
            pl.BlockSpec(memory_space=pltpu.MemorySpace.HBM),
            pl.BlockSpec(memory_space=pltpu.MemorySpace.HBM),
            pl.BlockSpec(memory_space=pltpu.MemorySpace.HBM),
        ],
        out_specs=pl.BlockSpec((T, D), lambda: (0, 0)),
        out_shape=jax.ShapeDtypeStruct((T, D), jnp.float32),
        scratch_shapes=[
            pltpu.VMEM((NBUF, G, F, D), jnp.float32),
            pltpu.VMEM((NBUF, G, F, D), jnp.float32),
            pltpu.VMEM((NBUF, G, D, F), jnp.float32),
            pltpu.SemaphoreType.DMA((NBUF,)),
            pltpu.SemaphoreType.DMA((NBUF,)),
            pltpu.SemaphoreType.DMA((NBUF,)),
        ],
        compiler_params=pltpu.CompilerParams(
            vmem_limit_bytes=60000 * 1024,
        ),
    )(x, route, w1, w3, w2)
